# baseline (speedup 1.0000x reference)
import functools

import jax
import jax.numpy as jnp
from jax import lax
from jax.experimental import pallas as pl
from jax.experimental.pallas import tpu as pltpu
from jax.experimental.pallas import tpu_sc as plsc

_NUM_WORKERS = 32
_CHUNK_ROWS = 32
_STREAM_BATCHES = (0, 1)
_SPMEM_BATCHES = (2, 3)


def _make_sc_copy(batch, seq_len, d_model, dtype):
    rows_per_w = seq_len // _NUM_WORKERS
    n_chunks = rows_per_w // _CHUNK_ROWS
    mesh = plsc.VectorSubcoreMesh(core_axis_name="c", subcore_axis_name="s")

    @functools.partial(
        pl.kernel,
        mesh=mesh,
        out_type=jax.ShapeDtypeStruct((batch, seq_len, d_model), dtype),
        scratch_types=[
            pltpu.VMEM((_CHUNK_ROWS, d_model), dtype),
            pltpu.VMEM((_CHUNK_ROWS, d_model), dtype),
            pltpu.VMEM_SHARED((_CHUNK_ROWS, d_model), dtype),
            pltpu.VMEM_SHARED((_CHUNK_ROWS, d_model), dtype),
            pltpu.SemaphoreType.DMA,
            pltpu.SemaphoreType.DMA,
            pltpu.SemaphoreType.DMA,
            pltpu.SemaphoreType.DMA,
            pltpu.SemaphoreType.DMA,
            pltpu.SemaphoreType.DMA,
        ],
    )
    def sc_copy(table_hbm, out_hbm, tbuf0, tbuf1, sbuf0, sbuf1,
                rsem, wsem0, wsem1, srsem, swsem0, swsem1):
        wid = lax.axis_index("s") * 2 + lax.axis_index("c")
        base = wid * rows_per_w
        tbufs = (tbuf0, tbuf1)
        sbufs = (sbuf0, sbuf1)
        wsems = (wsem0, wsem1)
        swsems = (swsem0, swsem1)
        treads = [None] * n_chunks
        sreads = [None] * n_chunks
        twrites = [[] for _ in range(n_chunks)]
        swrites = [[] for _ in range(n_chunks)]

        treads[0] = pltpu.async_copy(
            table_hbm.at[pl.ds(base, _CHUNK_ROWS)], tbuf0, rsem)
        sreads[0] = pltpu.async_copy(
            table_hbm.at[pl.ds(base, _CHUNK_ROWS)], sbuf0, srsem)
        for i in range(n_chunks):
            row0 = base + i * _CHUNK_ROWS
            nxt = base + (i + 1) * _CHUNK_ROWS
            treads[i].wait()
            for b in _STREAM_BATCHES:
                twrites[i].append(pltpu.async_copy(
                    tbufs[i % 2], out_hbm.at[b].at[pl.ds(row0, _CHUNK_ROWS)],
                    wsems[i % 2]))
            sreads[i].wait()
            for b in _SPMEM_BATCHES:
                swrites[i].append(pltpu.async_copy(
                    sbufs[i % 2], out_hbm.at[b].at[pl.ds(row0, _CHUNK_ROWS)],
                    swsems[i % 2]))
            if i + 1 < n_chunks:
                for h in (twrites[i - 1] + swrites[i - 1]) if i >= 1 else ():
                    h.wait()
                treads[i + 1] = pltpu.async_copy(
                    table_hbm.at[pl.ds(nxt, _CHUNK_ROWS)],
                    tbufs[(i + 1) % 2], rsem)
                sreads[i + 1] = pltpu.async_copy(
                    table_hbm.at[pl.ds(nxt, _CHUNK_ROWS)],
                    sbufs[(i + 1) % 2], srsem)
        for i in (n_chunks - 2, n_chunks - 1):
            if i >= 0:
                for h in twrites[i] + swrites[i]:
                    h.wait()

    return sc_copy


def kernel(input_ids, table):
    batch, seq_len = input_ids.shape
    max_pos, d_model = table.shape
    sc_copy = _make_sc_copy(batch, seq_len, d_model, table.dtype)
    return sc_copy(table)


# speedup vs baseline: 1.0026x; 1.0026x over previous
import functools

import jax
import jax.numpy as jnp
from jax import lax
from jax.experimental import pallas as pl
from jax.experimental.pallas import tpu as pltpu
from jax.experimental.pallas import tpu_sc as plsc

_NUM_WORKERS = 32
_NUM_SUBCORES = 16
_CHUNK_ROWS = 32
_STREAM_BATCHES = (0, 1)
_SPMEM_BATCHES = (2, 3)


def _make_sc_copy(batch, seq_len, d_model, dtype):
    rows_per_w = seq_len // _NUM_WORKERS
    n_chunks = rows_per_w // _CHUNK_ROWS
    mesh = plsc.VectorSubcoreMesh(core_axis_name="c", subcore_axis_name="s")

    @functools.partial(
        pl.kernel,
        mesh=mesh,
        out_type=jax.ShapeDtypeStruct((batch, seq_len, d_model), dtype),
        scratch_types=[
            pltpu.VMEM((_CHUNK_ROWS, d_model), dtype),
            pltpu.VMEM((_CHUNK_ROWS, d_model), dtype),
            pltpu.VMEM_SHARED((_NUM_SUBCORES * _CHUNK_ROWS, d_model), dtype),
            pltpu.VMEM_SHARED((_NUM_SUBCORES * _CHUNK_ROWS, d_model), dtype),
            pltpu.SemaphoreType.DMA,
            pltpu.SemaphoreType.DMA,
            pltpu.SemaphoreType.DMA,
            pltpu.SemaphoreType.DMA,
            pltpu.SemaphoreType.DMA,
            pltpu.SemaphoreType.DMA,
        ],
    )
    def sc_copy(table_hbm, out_hbm, tbuf0, tbuf1, sbuf0, sbuf1,
                rsem, wsem0, wsem1, srsem, swsem0, swsem1):
        sid = lax.axis_index("s")
        wid = sid * 2 + lax.axis_index("c")
        base = wid * rows_per_w
        ssl = pl.ds(sid * _CHUNK_ROWS, _CHUNK_ROWS)
        tbufs = (tbuf0, tbuf1)
        sbufs = (sbuf0.at[ssl], sbuf1.at[ssl])
        wsems = (wsem0, wsem1)
        swsems = (swsem0, swsem1)
        treads = [None] * n_chunks
        sreads = [None] * n_chunks
        twrites = [[] for _ in range(n_chunks)]
        swrites = [[] for _ in range(n_chunks)]

        treads[0] = pltpu.async_copy(
            table_hbm.at[pl.ds(base, _CHUNK_ROWS)], tbuf0, rsem)
        sreads[0] = pltpu.async_copy(
            table_hbm.at[pl.ds(base, _CHUNK_ROWS)], sbufs[0], srsem)
        for i in range(n_chunks):
            row0 = base + i * _CHUNK_ROWS
            nxt = base + (i + 1) * _CHUNK_ROWS
            treads[i].wait()
            for b in _STREAM_BATCHES:
                twrites[i].append(pltpu.async_copy(
                    tbufs[i % 2], out_hbm.at[b].at[pl.ds(row0, _CHUNK_ROWS)],
                    wsems[i % 2]))
            sreads[i].wait()
            for b in _SPMEM_BATCHES:
                swrites[i].append(pltpu.async_copy(
                    sbufs[i % 2], out_hbm.at[b].at[pl.ds(row0, _CHUNK_ROWS)],
                    swsems[i % 2]))
            if i + 1 < n_chunks:
                for h in (twrites[i - 1] + swrites[i - 1]) if i >= 1 else ():
                    h.wait()
                treads[i + 1] = pltpu.async_copy(
                    table_hbm.at[pl.ds(nxt, _CHUNK_ROWS)],
                    tbufs[(i + 1) % 2], rsem)
                sreads[i + 1] = pltpu.async_copy(
                    table_hbm.at[pl.ds(nxt, _CHUNK_ROWS)],
                    sbufs[(i + 1) % 2], srsem)
        for i in (n_chunks - 2, n_chunks - 1):
            if i >= 0:
                for h in twrites[i] + swrites[i]:
                    h.wait()

    return sc_copy


def kernel(input_ids, table):
    batch, seq_len = input_ids.shape
    max_pos, d_model = table.shape
    sc_copy = _make_sc_copy(batch, seq_len, d_model, table.dtype)
    return sc_copy(table)


# speedup vs baseline: 1.2530x; 1.2498x over previous
import functools

import jax
import jax.numpy as jnp
from jax import lax
from jax.experimental import pallas as pl
from jax.experimental.pallas import tpu as pltpu
from jax.experimental.pallas import tpu_sc as plsc

_NUM_WORKERS = 32
_CHUNK_ROWS = 64


def _make_sc_copy(batch, seq_len, d_model, dtype):
    rows_per_w = seq_len // _NUM_WORKERS
    n_chunks = rows_per_w // _CHUNK_ROWS
    mesh = plsc.VectorSubcoreMesh(core_axis_name="c", subcore_axis_name="s")

    @functools.partial(
        pl.kernel,
        mesh=mesh,
        out_type=jax.ShapeDtypeStruct((batch, seq_len, d_model), dtype),
        scratch_types=[
            pltpu.VMEM((_CHUNK_ROWS, d_model), dtype),
            pltpu.VMEM((_CHUNK_ROWS, d_model), dtype),
            pltpu.SemaphoreType.DMA,
            pltpu.SemaphoreType.DMA,
            pltpu.SemaphoreType.DMA,
        ],
    )
    def sc_copy(table_hbm, out_hbm, buf0, buf1, rsem, wsem0, wsem1):
        wid = lax.axis_index("c") * 16 + lax.axis_index("s")
        base = wid * rows_per_w
        bufs = (buf0, buf1)
        wsems = (wsem0, wsem1)
        reads = [None] * n_chunks
        writes = [[] for _ in range(n_chunks)]

        reads[0] = pltpu.async_copy(
            table_hbm.at[pl.ds(base, _CHUNK_ROWS)], buf0, rsem)
        for i in range(n_chunks):
            buf = bufs[i % 2]
            reads[i].wait()
            if i + 1 < n_chunks:
                for h in writes[i - 1] if i >= 1 else ():
                    h.wait()
                reads[i + 1] = pltpu.async_copy(
                    table_hbm.at[pl.ds(base + (i + 1) * _CHUNK_ROWS,
                                       _CHUNK_ROWS)],
                    bufs[(i + 1) % 2], rsem)
            row0 = base + i * _CHUNK_ROWS
            for b in range(batch):
                writes[i].append(pltpu.async_copy(
                    buf, out_hbm.at[b].at[pl.ds(row0, _CHUNK_ROWS)],
                    wsems[i % 2]))
        for i in (n_chunks - 2, n_chunks - 1):
            if i >= 0:
                for h in writes[i]:
                    h.wait()

    return sc_copy


def kernel(input_ids, table):
    batch, seq_len = input_ids.shape
    max_pos, d_model = table.shape
    sc_copy = _make_sc_copy(batch, seq_len, d_model, table.dtype)
    return sc_copy(table)
